# Initial kernel scaffold; baseline (speedup 1.0000x reference)
import jax
import jax.numpy as jnp
from jax.experimental import pallas as pl


def kernel(particles, adjacency_matrix, W1, b1, W2, b2, W3, b3, W4, b4):
    raise NotImplementedError("write your pallas kernel here")


# speedup vs baseline: 1.4563x; 1.4563x over previous
import jax
import jax.numpy as jnp
from jax.experimental import pallas as pl
from jax.experimental.pallas import tpu as pltpu

B, N, D = 4, 128, 128
HID, REL = 256, 64
TI = 16
K = N // TI


def _fused_body(p_ref, a_ref, w1_ref, b1_ref, w2_ref, b2_ref,
                w3_ref, b3_ref, w4_ref, b4_ref,
                out_ref, s_ref, r_ref, acc_ref):
    k = pl.program_id(1)

    @pl.when(k == 0)
    def _init():
        p = p_ref[0]
        s_ref[...] = jnp.dot(p, w1_ref[:D, :], preferred_element_type=jnp.float32)
        r_ref[...] = jnp.dot(p, w1_ref[D:, :], preferred_element_type=jnp.float32)
        acc_ref[...] = jnp.zeros_like(acc_ref)

    s_blk = s_ref[pl.ds(k * TI, TI), :]
    h = jnp.maximum(
        s_blk[:, None, :] + r_ref[...][None, :, :] + b1_ref[0][None, None, :],
        0.0)
    f = jnp.dot(h.reshape(TI * N, HID), w2_ref[...],
                preferred_element_type=jnp.float32) + b2_ref[0][None, :]
    f = jnp.maximum(f, 0.0).reshape(TI, N, REL)
    acc_ref[...] += jnp.sum(f * a_ref[...][:, :, None], axis=0)

    @pl.when(k == K - 1)
    def _final():
        p = p_ref[0]
        h2 = jnp.maximum(
            jnp.dot(p, w3_ref[:D, :], preferred_element_type=jnp.float32)
            + jnp.dot(acc_ref[...], w3_ref[D:, :], preferred_element_type=jnp.float32)
            + b3_ref[0][None, :],
            0.0)
        delta = jnp.dot(h2, w4_ref[...], preferred_element_type=jnp.float32) \
            + b4_ref[0][None, :]
        out_ref[0] = p + delta


def kernel(particles, adjacency_matrix, W1, b1, W2, b2, W3, b3, W4, b4):
    mask = (adjacency_matrix == 1).astype(jnp.float32)
    b1r = b1.reshape(1, HID)
    b2r = b2.reshape(1, REL)
    b3r = b3.reshape(1, HID)
    b4r = b4.reshape(1, D)
    full = lambda shape: pl.BlockSpec(shape, lambda b, k: (0,) * len(shape))
    return pl.pallas_call(
        _fused_body,
        grid=(B, K),
        in_specs=[
            pl.BlockSpec((1, N, D), lambda b, k: (b, 0, 0)),
            pl.BlockSpec((TI, N), lambda b, k: (k, 0)),
            full((2 * D, HID)),
            full((1, HID)),
            full((HID, REL)),
            full((1, REL)),
            full((D + REL, HID)),
            full((1, HID)),
            full((HID, D)),
            full((1, D)),
        ],
        out_specs=pl.BlockSpec((1, N, D), lambda b, k: (b, 0, 0)),
        out_shape=jax.ShapeDtypeStruct((B, N, D), jnp.float32),
        scratch_shapes=[
            pltpu.VMEM((N, HID), jnp.float32),
            pltpu.VMEM((N, HID), jnp.float32),
            pltpu.VMEM((N, REL), jnp.float32),
        ],
    )(particles, mask, W1, b1r, W2, b2r, W3, b3r, W4, b4r)


# speedup vs baseline: 1.4874x; 1.0214x over previous
import jax
import jax.numpy as jnp
from jax.experimental import pallas as pl
from jax.experimental.pallas import tpu as pltpu

B, N, D = 4, 128, 128
HID, REL = 256, 64
TI = 16
K = N // TI


def _fused_body(p_ref, a_ref, w1_ref, b1_ref, w2_ref, b2_ref,
                w3_ref, b3_ref, w4_ref, b4_ref,
                out_ref, s_ref, r_ref, acc_ref):
    k = pl.program_id(1)

    @pl.when(k == 0)
    def _init():
        p = p_ref[0]
        s_ref[...] = (jnp.dot(p, w1_ref[:D, :], preferred_element_type=jnp.float32)
                      + b1_ref[0][None, :]).astype(jnp.bfloat16)
        r_ref[...] = jnp.dot(p, w1_ref[D:, :],
                             preferred_element_type=jnp.float32).astype(jnp.bfloat16)
        acc_ref[...] = jnp.zeros_like(acc_ref)

    s_blk = s_ref[pl.ds(k * TI, TI), :]
    h = jnp.maximum(s_blk[:, None, :] + r_ref[...][None, :, :],
                    jnp.bfloat16(0.0))
    f = jnp.dot(h.reshape(TI * N, HID), w2_ref[...],
                preferred_element_type=jnp.float32) + b2_ref[0][None, :]
    f = jnp.maximum(f, 0.0).reshape(TI, N, REL)
    acc_ref[...] += jnp.sum(f * a_ref[...][:, :, None], axis=0)

    @pl.when(k == K - 1)
    def _final():
        p = p_ref[0]
        h2 = jnp.maximum(
            jnp.dot(p, w3_ref[:D, :], preferred_element_type=jnp.float32)
            + jnp.dot(acc_ref[...], w3_ref[D:, :], preferred_element_type=jnp.float32)
            + b3_ref[0][None, :],
            0.0)
        delta = jnp.dot(h2, w4_ref[...], preferred_element_type=jnp.float32) \
            + b4_ref[0][None, :]
        out_ref[0] = p + delta


def kernel(particles, adjacency_matrix, W1, b1, W2, b2, W3, b3, W4, b4):
    mask = (adjacency_matrix == 1).astype(jnp.float32)
    W2 = W2.astype(jnp.bfloat16)
    b1r = b1.reshape(1, HID)
    b2r = b2.reshape(1, REL)
    b3r = b3.reshape(1, HID)
    b4r = b4.reshape(1, D)
    full = lambda shape: pl.BlockSpec(shape, lambda b, k: (0,) * len(shape))
    return pl.pallas_call(
        _fused_body,
        grid=(B, K),
        in_specs=[
            pl.BlockSpec((1, N, D), lambda b, k: (b, 0, 0)),
            pl.BlockSpec((TI, N), lambda b, k: (k, 0)),
            full((2 * D, HID)),
            full((1, HID)),
            full((HID, REL)),
            full((1, REL)),
            full((D + REL, HID)),
            full((1, HID)),
            full((HID, D)),
            full((1, D)),
        ],
        out_specs=pl.BlockSpec((1, N, D), lambda b, k: (b, 0, 0)),
        out_shape=jax.ShapeDtypeStruct((B, N, D), jnp.float32),
        scratch_shapes=[
            pltpu.VMEM((N, HID), jnp.bfloat16),
            pltpu.VMEM((N, HID), jnp.bfloat16),
            pltpu.VMEM((N, REL), jnp.float32),
        ],
    )(particles, mask, W1, b1r, W2, b2r, W3, b3r, W4, b4r)


# speedup vs baseline: 1.9416x; 1.3054x over previous
import jax
import jax.numpy as jnp
from jax.experimental import pallas as pl
from jax.experimental.pallas import tpu as pltpu

B, N, D = 4, 128, 128
HID, REL = 256, 64
TI = 32
SUB = 16
K = N // TI


def _fused_body(p_ref, a_ref, w1_ref, b1_ref, w2_ref, b2_ref,
                w3_ref, b3_ref, w4_ref, b4_ref,
                out_ref, s_ref, r_ref, acc_ref):
    k = pl.program_id(1)

    @pl.when(k == 0)
    def _init():
        p = p_ref[0]
        s_ref[...] = (jnp.dot(p, w1_ref[:D, :], preferred_element_type=jnp.float32)
                      + b1_ref[0][None, :]).astype(jnp.bfloat16)
        r_ref[...] = jnp.dot(p, w1_ref[D:, :],
                             preferred_element_type=jnp.float32).astype(jnp.bfloat16)
        acc_ref[...] = jnp.zeros_like(acc_ref)

    r_all = r_ref[...]
    contribs = []
    for u in range(TI // SUB):
        s_blk = s_ref[pl.ds(k * TI + u * SUB, SUB), :]
        h = jnp.maximum(s_blk[:, None, :] + r_all[None, :, :],
                        jnp.bfloat16(0.0))
        f = jnp.dot(h.reshape(SUB * N, HID), w2_ref[...],
                    preferred_element_type=jnp.float32) + b2_ref[0][None, :]
        f = jnp.maximum(f, 0.0).reshape(SUB, N, REL)
        a_blk = a_ref[pl.ds(u * SUB, SUB), :]
        contribs.append(jnp.sum(f * a_blk[:, :, None], axis=0))
    acc_ref[...] += sum(contribs)

    @pl.when(k == K - 1)
    def _final():
        p = p_ref[0]
        h2 = jnp.maximum(
            jnp.dot(p, w3_ref[:D, :], preferred_element_type=jnp.float32)
            + jnp.dot(acc_ref[...], w3_ref[D:, :], preferred_element_type=jnp.float32)
            + b3_ref[0][None, :],
            0.0)
        delta = jnp.dot(h2, w4_ref[...], preferred_element_type=jnp.float32) \
            + b4_ref[0][None, :]
        out_ref[0] = p + delta


def kernel(particles, adjacency_matrix, W1, b1, W2, b2, W3, b3, W4, b4):
    mask = (adjacency_matrix == 1).astype(jnp.float32)
    W2 = W2.astype(jnp.bfloat16)
    b1r = b1.reshape(1, HID)
    b2r = b2.reshape(1, REL)
    b3r = b3.reshape(1, HID)
    b4r = b4.reshape(1, D)
    full = lambda shape: pl.BlockSpec(shape, lambda b, k: (0,) * len(shape))
    return pl.pallas_call(
        _fused_body,
        grid=(B, K),
        in_specs=[
            pl.BlockSpec((1, N, D), lambda b, k: (b, 0, 0)),
            pl.BlockSpec((TI, N), lambda b, k: (k, 0)),
            full((2 * D, HID)),
            full((1, HID)),
            full((HID, REL)),
            full((1, REL)),
            full((D + REL, HID)),
            full((1, HID)),
            full((HID, D)),
            full((1, D)),
        ],
        out_specs=pl.BlockSpec((1, N, D), lambda b, k: (b, 0, 0)),
        out_shape=jax.ShapeDtypeStruct((B, N, D), jnp.float32),
        scratch_shapes=[
            pltpu.VMEM((N, HID), jnp.bfloat16),
            pltpu.VMEM((N, HID), jnp.bfloat16),
            pltpu.VMEM((N, REL), jnp.float32),
        ],
    )(particles, mask, W1, b1r, W2, b2r, W3, b3r, W4, b4r)


# speedup vs baseline: 2.1768x; 1.1211x over previous
import jax
import jax.numpy as jnp
from jax.experimental import pallas as pl
from jax.experimental.pallas import tpu as pltpu

B, N, D = 4, 128, 128
HID, REL = 256, 64
TI = 64
SUB = 16
K = N // TI


def _fused_body(p_ref, a_ref, w1_ref, b1_ref, w2_ref, b2_ref,
                w3_ref, b3_ref, w4_ref, b4_ref,
                out_ref, s_ref, r_ref, acc_ref):
    k = pl.program_id(1)

    @pl.when(k == 0)
    def _init():
        p = p_ref[0]
        s_ref[...] = (jnp.dot(p, w1_ref[:D, :], preferred_element_type=jnp.float32)
                      + b1_ref[0][None, :]).astype(jnp.bfloat16)
        r_ref[...] = jnp.dot(p, w1_ref[D:, :],
                             preferred_element_type=jnp.float32).astype(jnp.bfloat16)
        acc_ref[...] = jnp.zeros_like(acc_ref)

    r_all = r_ref[...]
    contribs = []
    for u in range(TI // SUB):
        s_blk = s_ref[pl.ds(k * TI + u * SUB, SUB), :]
        h = jnp.maximum(s_blk[:, None, :] + r_all[None, :, :],
                        jnp.bfloat16(0.0))
        f = jnp.dot(h.reshape(SUB * N, HID), w2_ref[...],
                    preferred_element_type=jnp.float32) + b2_ref[0][None, :]
        f = jnp.maximum(f, 0.0).reshape(SUB, N, REL)
        a_blk = a_ref[pl.ds(u * SUB, SUB), :]
        contribs.append(jnp.sum(f * a_blk[:, :, None], axis=0))
    acc_ref[...] += sum(contribs)

    @pl.when(k == K - 1)
    def _final():
        p = p_ref[0]
        h2 = jnp.maximum(
            jnp.dot(p, w3_ref[:D, :], preferred_element_type=jnp.float32)
            + jnp.dot(acc_ref[...], w3_ref[D:, :], preferred_element_type=jnp.float32)
            + b3_ref[0][None, :],
            0.0)
        delta = jnp.dot(h2, w4_ref[...], preferred_element_type=jnp.float32) \
            + b4_ref[0][None, :]
        out_ref[0] = p + delta


def kernel(particles, adjacency_matrix, W1, b1, W2, b2, W3, b3, W4, b4):
    mask = (adjacency_matrix == 1).astype(jnp.float32)
    W2 = W2.astype(jnp.bfloat16)
    b1r = b1.reshape(1, HID)
    b2r = b2.reshape(1, REL)
    b3r = b3.reshape(1, HID)
    b4r = b4.reshape(1, D)
    full = lambda shape: pl.BlockSpec(shape, lambda b, k: (0,) * len(shape))
    return pl.pallas_call(
        _fused_body,
        grid=(B, K),
        in_specs=[
            pl.BlockSpec((1, N, D), lambda b, k: (b, 0, 0)),
            pl.BlockSpec((TI, N), lambda b, k: (k, 0)),
            full((2 * D, HID)),
            full((1, HID)),
            full((HID, REL)),
            full((1, REL)),
            full((D + REL, HID)),
            full((1, HID)),
            full((HID, D)),
            full((1, D)),
        ],
        out_specs=pl.BlockSpec((1, N, D), lambda b, k: (b, 0, 0)),
        out_shape=jax.ShapeDtypeStruct((B, N, D), jnp.float32),
        scratch_shapes=[
            pltpu.VMEM((N, HID), jnp.bfloat16),
            pltpu.VMEM((N, HID), jnp.bfloat16),
            pltpu.VMEM((N, REL), jnp.float32),
        ],
    )(particles, mask, W1, b1r, W2, b2r, W3, b3r, W4, b4r)


# speedup vs baseline: 2.4482x; 1.1247x over previous
import jax
import jax.numpy as jnp
from jax.experimental import pallas as pl
from jax.experimental.pallas import tpu as pltpu

B, N, D = 4, 128, 128
HID, REL = 256, 64
TI = 128
SUB = 16
K = N // TI


def _fused_body(p_ref, a_ref, w1_ref, b1_ref, w2_ref, b2_ref,
                w3_ref, b3_ref, w4_ref, b4_ref,
                out_ref, s_ref, r_ref, acc_ref):
    k = pl.program_id(1)

    @pl.when(k == 0)
    def _init():
        p = p_ref[0]
        s_ref[...] = (jnp.dot(p, w1_ref[:D, :], preferred_element_type=jnp.float32)
                      + b1_ref[0][None, :]).astype(jnp.bfloat16)
        r_ref[...] = jnp.dot(p, w1_ref[D:, :],
                             preferred_element_type=jnp.float32).astype(jnp.bfloat16)
        acc_ref[...] = jnp.zeros_like(acc_ref)

    r_all = r_ref[...]
    contribs = []
    for u in range(TI // SUB):
        s_blk = s_ref[pl.ds(k * TI + u * SUB, SUB), :]
        h = jnp.maximum(s_blk[:, None, :] + r_all[None, :, :],
                        jnp.bfloat16(0.0))
        f = jnp.dot(h.reshape(SUB * N, HID), w2_ref[...],
                    preferred_element_type=jnp.float32) + b2_ref[0][None, :]
        f = jnp.maximum(f, 0.0).reshape(SUB, N, REL)
        a_blk = a_ref[pl.ds(u * SUB, SUB), :]
        contribs.append(jnp.sum(f * a_blk[:, :, None], axis=0))
    acc_ref[...] += sum(contribs)

    @pl.when(k == K - 1)
    def _final():
        p = p_ref[0]
        h2 = jnp.maximum(
            jnp.dot(p, w3_ref[:D, :], preferred_element_type=jnp.float32)
            + jnp.dot(acc_ref[...], w3_ref[D:, :], preferred_element_type=jnp.float32)
            + b3_ref[0][None, :],
            0.0)
        delta = jnp.dot(h2, w4_ref[...], preferred_element_type=jnp.float32) \
            + b4_ref[0][None, :]
        out_ref[0] = p + delta


def kernel(particles, adjacency_matrix, W1, b1, W2, b2, W3, b3, W4, b4):
    mask = (adjacency_matrix == 1).astype(jnp.float32)
    W2 = W2.astype(jnp.bfloat16)
    b1r = b1.reshape(1, HID)
    b2r = b2.reshape(1, REL)
    b3r = b3.reshape(1, HID)
    b4r = b4.reshape(1, D)
    full = lambda shape: pl.BlockSpec(shape, lambda b, k: (0,) * len(shape))
    return pl.pallas_call(
        _fused_body,
        grid=(B, K),
        in_specs=[
            pl.BlockSpec((1, N, D), lambda b, k: (b, 0, 0)),
            pl.BlockSpec((TI, N), lambda b, k: (k, 0)),
            full((2 * D, HID)),
            full((1, HID)),
            full((HID, REL)),
            full((1, REL)),
            full((D + REL, HID)),
            full((1, HID)),
            full((HID, D)),
            full((1, D)),
        ],
        out_specs=pl.BlockSpec((1, N, D), lambda b, k: (b, 0, 0)),
        out_shape=jax.ShapeDtypeStruct((B, N, D), jnp.float32),
        scratch_shapes=[
            pltpu.VMEM((N, HID), jnp.bfloat16),
            pltpu.VMEM((N, HID), jnp.bfloat16),
            pltpu.VMEM((N, REL), jnp.float32),
        ],
    )(particles, mask, W1, b1r, W2, b2r, W3, b3r, W4, b4r)


# speedup vs baseline: 2.7499x; 1.1232x over previous
import jax
import jax.numpy as jnp
from jax.experimental import pallas as pl
from jax.experimental.pallas import tpu as pltpu

B, N, D = 4, 128, 128
HID, REL = 256, 64
SUB = 16


def _fused_body(p_ref, a_ref, w1_ref, b1_ref, w2_ref, b2_ref,
                w3_ref, b3_ref, w4_ref, b4_ref, out_ref):
    p_all = p_ref[...]
    s_all = (jnp.dot(p_all, w1_ref[:D, :], preferred_element_type=jnp.float32)
             + b1_ref[0][None, :]).astype(jnp.bfloat16)
    r_all = jnp.dot(p_all, w1_ref[D:, :],
                    preferred_element_type=jnp.float32).astype(jnp.bfloat16)

    rel_blocks = []
    for b in range(B):
        r_b = r_all[b * N:(b + 1) * N, :]
        contribs = []
        for u in range(N // SUB):
            lo = b * N + u * SUB
            s_blk = s_all[lo:lo + SUB, :]
            h = jnp.maximum(s_blk[:, None, :] + r_b[None, :, :],
                            jnp.bfloat16(0.0))
            f = jnp.dot(h.reshape(SUB * N, HID), w2_ref[...],
                        preferred_element_type=jnp.float32) + b2_ref[0][None, :]
            f = jnp.maximum(f, 0.0).reshape(SUB, N, REL)
            a_blk = a_ref[u * SUB:(u + 1) * SUB, :]
            contribs.append(jnp.sum(f * a_blk[:, :, None], axis=0))
        rel_blocks.append(sum(contribs))
    rel = jnp.concatenate(rel_blocks, axis=0)

    h2 = jnp.maximum(
        jnp.dot(p_all, w3_ref[:D, :], preferred_element_type=jnp.float32)
        + jnp.dot(rel, w3_ref[D:, :], preferred_element_type=jnp.float32)
        + b3_ref[0][None, :],
        0.0)
    delta = jnp.dot(h2, w4_ref[...], preferred_element_type=jnp.float32) \
        + b4_ref[0][None, :]
    out_ref[...] = p_all + delta


def kernel(particles, adjacency_matrix, W1, b1, W2, b2, W3, b3, W4, b4):
    mask = (adjacency_matrix == 1).astype(jnp.float32)
    out = pl.pallas_call(
        _fused_body,
        out_shape=jax.ShapeDtypeStruct((B * N, D), jnp.float32),
    )(particles.reshape(B * N, D), mask, W1, b1.reshape(1, HID),
      W2.astype(jnp.bfloat16), b2.reshape(1, REL),
      W3, b3.reshape(1, HID), W4, b4.reshape(1, D))
    return out.reshape(B, N, D)
